# baseline (speedup 1.0000x reference)
import jax
import jax.numpy as jnp
from jax import lax
from jax.experimental import pallas as pl
from jax.experimental.pallas import tpu as pltpu
from jax.experimental.pallas import tpu_sc as plsc

N = 10000
E = 320000
D = 128

NC = 2
NS = 16
NW = NC * NS
EPW = E // NW
CH = 128
NCH = 79
EPWP = NCH * CH
PAD = EPWP - EPW
NP = 10112
RPT = NP // NS
CNP = 10240
CPT = 1280
DH = D // 2
EPT = E // NS
NCHA = 158
PADA = NCHA * CH - EPT

_mesh = lambda: plsc.VectorSubcoreMesh(core_axis_name="c", subcore_axis_name="s")


def _sc_agg(y_st, src, dst, z2):
    def body(y_r, src_r, dst_r, z2_r, s_out_r,
             src_v, dst_v, rows0, rows1, shared_s, gsem0, gsem1, ssem0,
             ssem1):
        c = lax.axis_index("c")
        s = lax.axis_index("s")
        yc = y_r.at[c]

        pltpu.sync_copy(src_r.at[s], src_v)
        pltpu.sync_copy(dst_r.at[s], dst_v)
        pltpu.sync_copy(z2_r, shared_s.at[pl.ds(s * RPT, RPT)])
        plsc.subcore_barrier()

        def chunk(i, carry):
            j0 = i * 2
            j1 = i * 2 + 1
            pltpu.async_copy(yc.at[src_v.at[j0]], rows0, gsem0).wait()
            s0 = pltpu.async_copy(rows0, shared_s.at[dst_v.at[j0]], ssem0,
                                  add=True)
            pltpu.async_copy(yc.at[src_v.at[j1]], rows1, gsem1).wait()
            s1 = pltpu.async_copy(rows1, shared_s.at[dst_v.at[j1]], ssem1,
                                  add=True)
            s0.wait()
            s1.wait()
            return carry

        lax.fori_loop(0, NCHA // 2, chunk, 0)
        plsc.subcore_barrier()

        pltpu.sync_copy(shared_s.at[pl.ds(s * RPT, RPT)],
                        s_out_r.at[c, pl.ds(s * RPT, RPT)])

    return pl.kernel(
        body,
        out_type=jax.ShapeDtypeStruct((NC, NP, DH), jnp.float32),
        mesh=_mesh(),
        compiler_params=pltpu.CompilerParams(use_tc_tiling_on_sc=False),
        scratch_types=[
            pltpu.VMEM((NCHA, CH), jnp.int32),
            pltpu.VMEM((NCHA, CH), jnp.int32),
            pltpu.VMEM((CH, DH), jnp.float32),
            pltpu.VMEM((CH, DH), jnp.float32),
            pltpu.VMEM_SHARED((NP, DH), jnp.float32),
            pltpu.SemaphoreType.DMA,
            pltpu.SemaphoreType.DMA,
            pltpu.SemaphoreType.DMA,
            pltpu.SemaphoreType.DMA,
        ],
    )(y_st, src, dst, z2)


def _sc_cnt(dst, z1):
    def body(dst_r, z1_r, cnt_out_r, dst_v, ones_v, shared_cnt):
        c = lax.axis_index("c")
        s = lax.axis_index("s")
        wid = s * NC + c

        pltpu.sync_copy(dst_r.at[wid], dst_v)
        @pl.when(s < CNP // CPT)
        def _():
            pltpu.sync_copy(z1_r, shared_cnt.at[pl.ds(s * CPT, CPT)])
        for k in range(CH // 16):
            ones_v[pl.ds(k * 16, 16)] = jnp.full((16,), 1.0, jnp.float32)
        plsc.subcore_barrier()

        def chunk(j, carry):
            pltpu.sync_copy(ones_v, shared_cnt.at[dst_v.at[j]], add=True)
            return carry

        lax.fori_loop(0, NCH, chunk, 0)
        plsc.subcore_barrier()

        @pl.when(s < CNP // CPT)
        def _():
            pltpu.sync_copy(shared_cnt.at[pl.ds(s * CPT, CPT)],
                            cnt_out_r.at[c, pl.ds(s * CPT, CPT)])

    return pl.kernel(
        body,
        out_type=jax.ShapeDtypeStruct((NC, CNP), jnp.float32),
        mesh=_mesh(),
        scratch_types=[
            pltpu.VMEM((NCH, CH), jnp.int32),
            pltpu.VMEM((CH,), jnp.float32),
            pltpu.VMEM_SHARED((CNP,), jnp.float32),
        ],
    )(dst, z1)


ROWS_B = 2000


def _mm_body(x_r, w_r, o_r):
    o_r[0] = jnp.dot(x_r[...], w_r[0], preferred_element_type=jnp.float32)


def _tc_mm(x, w):
    w_st = jnp.stack([w[:, :DH], w[:, DH:]])
    return pl.pallas_call(
        _mm_body,
        grid=(N // ROWS_B, NC),
        in_specs=[pl.BlockSpec((ROWS_B, D), lambda i, j: (i, 0)),
                  pl.BlockSpec((1, D, DH), lambda i, j: (j, 0, 0))],
        out_specs=pl.BlockSpec((1, ROWS_B, DH), lambda i, j: (j, i, 0)),
        out_shape=jax.ShapeDtypeStruct((NC, N, DH), jnp.float32),
    )(x, w_st)


def _tc_mid_body(s_r, ct_r, x_r, wr_r, b_r, wl2_r, h_r, y2_r):
    tot = ct_r[:, 0:1] + ct_r[:, 1:2]
    inv = 1.0 / jnp.maximum(tot, 1.0)
    agg = jnp.concatenate([s_r[0], s_r[1]], axis=-1) * inv
    xw = jnp.dot(x_r[...], wr_r[...], preferred_element_type=jnp.float32)
    h = jnp.maximum(agg + xw + b_r[...], 0.0)
    h_r[...] = h
    y2 = jnp.dot(h, wl2_r[...], preferred_element_type=jnp.float32)
    y2_r[0] = y2[:, :DH]
    y2_r[1] = y2[:, DH:]


def _tc_mid(s1, cnt_t, x, wr, b, wl2):
    return pl.pallas_call(
        _tc_mid_body,
        grid=(N // ROWS_B,),
        in_specs=[pl.BlockSpec((NC, ROWS_B, DH), lambda i: (0, i, 0)),
                  pl.BlockSpec((ROWS_B, NC), lambda i: (i, 0)),
                  pl.BlockSpec((ROWS_B, D), lambda i: (i, 0)),
                  pl.BlockSpec((D, D), lambda i: (0, 0)),
                  pl.BlockSpec((1, D), lambda i: (0, 0)),
                  pl.BlockSpec((D, D), lambda i: (0, 0))],
        out_specs=(pl.BlockSpec((ROWS_B, D), lambda i: (i, 0)),
                   pl.BlockSpec((NC, ROWS_B, DH), lambda i: (0, i, 0))),
        out_shape=(jax.ShapeDtypeStruct((N, D), jnp.float32),
                   jax.ShapeDtypeStruct((NC, N, DH), jnp.float32)),
    )(s1, cnt_t, x, wr, b, wl2)


def _tc_out_body(s_r, ct_r, h_r, wr_r, b_r, o_r):
    tot = ct_r[:, 0:1] + ct_r[:, 1:2]
    inv = 1.0 / jnp.maximum(tot, 1.0)
    agg = jnp.concatenate([s_r[0], s_r[1]], axis=-1) * inv
    hw = jnp.dot(h_r[...], wr_r[...], preferred_element_type=jnp.float32)
    o_r[...] = agg + hw + b_r[...]


def _tc_out(s2, cnt_t, h, wr, b):
    return pl.pallas_call(
        _tc_out_body,
        grid=(N // ROWS_B,),
        in_specs=[pl.BlockSpec((NC, ROWS_B, DH), lambda i: (0, i, 0)),
                  pl.BlockSpec((ROWS_B, NC), lambda i: (i, 0)),
                  pl.BlockSpec((ROWS_B, D), lambda i: (i, 0)),
                  pl.BlockSpec((D, D), lambda i: (0, 0)),
                  pl.BlockSpec((1, D), lambda i: (0, 0))],
        out_specs=pl.BlockSpec((ROWS_B, D), lambda i: (i, 0)),
        out_shape=jax.ShapeDtypeStruct((N, D), jnp.float32),
    )(s2, cnt_t, h, wr, b)


def _pad_edges(idx, nshard, per, pad_base, nch):
    npad = nch * CH - per
    tiles = idx.astype(jnp.int32).reshape(nshard, per)
    padv = pad_base + jnp.arange(npad, dtype=jnp.int32)
    pad = jnp.broadcast_to(padv, (nshard, npad))
    return jnp.concatenate([tiles, pad], axis=1).reshape(nshard, nch, CH)


def kernel(x, edge_index, W_l1, W_r1, b1, W_l2, W_r2, b2):
    srcA = _pad_edges(edge_index[0], NS, EPT, 0, NCHA)
    dstA = _pad_edges(edge_index[1], NS, EPT, N, NCHA)
    dstC = _pad_edges(edge_index[1], NW, EPW, N, NCH)
    z2 = jnp.zeros((RPT, DH), jnp.float32)
    z1 = jnp.zeros((CPT,), jnp.float32)

    cnt = _sc_cnt(dstC, z1)
    cnt_t = cnt.T
    y1 = _tc_mm(x, W_l1)
    s1 = _sc_agg(y1, srcA, dstA, z2)
    h, y2 = _tc_mid(s1, cnt_t, x, W_r1, b1.reshape(1, D), W_l2)
    s2 = _sc_agg(y2, srcA, dstA, z2)
    return _tc_out(s2, cnt_t, h, W_r2, b2.reshape(1, D))


# speedup vs baseline: 1.2004x; 1.2004x over previous
import jax
import jax.numpy as jnp
from jax import lax
from jax.experimental import pallas as pl
from jax.experimental.pallas import tpu as pltpu
from jax.experimental.pallas import tpu_sc as plsc

N = 10000
E = 320000
D = 128

NC = 2
NS = 16
NW = NC * NS
EPW = E // NW
CH = 128
NCH = 79
EPWP = NCH * CH
PAD = EPWP - EPW
NP = 10112
RPT = NP // NS
CNP = 10240
CPT = 1280

_mesh = lambda: plsc.VectorSubcoreMesh(core_axis_name="c", subcore_axis_name="s")


def _sc_agg(y, src, dst, z2):
    def body(y_r, src_r, dst_r, z2_r, s_out_r,
             src_v, dst_v, rows, shared_s, gsem):
        c = lax.axis_index("c")
        s = lax.axis_index("s")
        wid = s * NC + c

        pltpu.sync_copy(src_r.at[wid], src_v)
        pltpu.sync_copy(dst_r.at[wid], dst_v)
        pltpu.sync_copy(z2_r, shared_s.at[pl.ds(s * RPT, RPT)])
        plsc.subcore_barrier()

        def chunk(i, carry):
            j0 = i * 2
            j1 = i * 2 + 1
            pltpu.async_copy(y_r.at[src_v.at[j0]], rows, gsem).wait()
            pltpu.sync_copy(rows, shared_s.at[dst_v.at[j0]], add=True)
            pltpu.async_copy(y_r.at[src_v.at[j1]], rows, gsem).wait()
            pltpu.sync_copy(rows, shared_s.at[dst_v.at[j1]], add=True)
            return carry

        lax.fori_loop(0, NCH // 2, chunk, 0)
        j = NCH - 1
        pltpu.async_copy(y_r.at[src_v.at[j]], rows, gsem).wait()
        pltpu.sync_copy(rows, shared_s.at[dst_v.at[j]], add=True)
        plsc.subcore_barrier()

        pltpu.sync_copy(shared_s.at[pl.ds(s * RPT, RPT)],
                        s_out_r.at[c, pl.ds(s * RPT, RPT)])

    return pl.kernel(
        body,
        out_type=jax.ShapeDtypeStruct((NC, NP, D), jnp.float32),
        mesh=_mesh(),
        scratch_types=[
            pltpu.VMEM((NCH, CH), jnp.int32),
            pltpu.VMEM((NCH, CH), jnp.int32),
            pltpu.VMEM((CH, D), jnp.float32),
            pltpu.VMEM_SHARED((NP, D), jnp.float32),
            pltpu.SemaphoreType.DMA,
        ],
    )(y, src, dst, z2)


def _sc_cnt(dst, z1):
    def body(dst_r, z1_r, cnt_out_r, dst_v, ones_v, shared_cnt):
        c = lax.axis_index("c")
        s = lax.axis_index("s")
        wid = s * NC + c

        pltpu.sync_copy(dst_r.at[wid], dst_v)
        @pl.when(s < CNP // CPT)
        def _():
            pltpu.sync_copy(z1_r, shared_cnt.at[pl.ds(s * CPT, CPT)])
        for k in range(CH // 16):
            ones_v[pl.ds(k * 16, 16)] = jnp.full((16,), 1.0, jnp.float32)
        plsc.subcore_barrier()

        def chunk(j, carry):
            pltpu.sync_copy(ones_v, shared_cnt.at[dst_v.at[j]], add=True)
            return carry

        lax.fori_loop(0, NCH, chunk, 0)
        plsc.subcore_barrier()

        @pl.when(s < CNP // CPT)
        def _():
            pltpu.sync_copy(shared_cnt.at[pl.ds(s * CPT, CPT)],
                            cnt_out_r.at[c, pl.ds(s * CPT, CPT)])

    return pl.kernel(
        body,
        out_type=jax.ShapeDtypeStruct((NC, CNP), jnp.float32),
        mesh=_mesh(),
        scratch_types=[
            pltpu.VMEM((NCH, CH), jnp.int32),
            pltpu.VMEM((CH,), jnp.float32),
            pltpu.VMEM_SHARED((CNP,), jnp.float32),
        ],
    )(dst, z1)


ROWS_B = 2000


def _mm_body(x_r, w_r, o_r):
    o_r[...] = jnp.dot(x_r[...], w_r[...], preferred_element_type=jnp.float32)


def _tc_mm(x, w):
    return pl.pallas_call(
        _mm_body,
        grid=(N // ROWS_B,),
        in_specs=[pl.BlockSpec((ROWS_B, D), lambda i: (i, 0)),
                  pl.BlockSpec((D, D), lambda i: (0, 0))],
        out_specs=pl.BlockSpec((ROWS_B, D), lambda i: (i, 0)),
        out_shape=jax.ShapeDtypeStruct((N, D), jnp.float32),
    )(x, w)


def _tc_mid_body(s_r, ct_r, x_r, wr_r, b_r, wl2_r, h_r, y2_r):
    tot = ct_r[:, 0:1] + ct_r[:, 1:2]
    inv = 1.0 / jnp.maximum(tot, 1.0)
    agg = (s_r[0] + s_r[1]) * inv
    xw = jnp.dot(x_r[...], wr_r[...], preferred_element_type=jnp.float32)
    h = jnp.maximum(agg + xw + b_r[...], 0.0)
    h_r[...] = h
    y2_r[...] = jnp.dot(h, wl2_r[...], preferred_element_type=jnp.float32)


def _tc_mid(s1, cnt_t, x, wr, b, wl2):
    return pl.pallas_call(
        _tc_mid_body,
        grid=(N // ROWS_B,),
        in_specs=[pl.BlockSpec((NC, ROWS_B, D), lambda i: (0, i, 0)),
                  pl.BlockSpec((ROWS_B, NC), lambda i: (i, 0)),
                  pl.BlockSpec((ROWS_B, D), lambda i: (i, 0)),
                  pl.BlockSpec((D, D), lambda i: (0, 0)),
                  pl.BlockSpec((1, D), lambda i: (0, 0)),
                  pl.BlockSpec((D, D), lambda i: (0, 0))],
        out_specs=(pl.BlockSpec((ROWS_B, D), lambda i: (i, 0)),
                   pl.BlockSpec((ROWS_B, D), lambda i: (i, 0))),
        out_shape=(jax.ShapeDtypeStruct((N, D), jnp.float32),
                   jax.ShapeDtypeStruct((N, D), jnp.float32)),
    )(s1, cnt_t, x, wr, b, wl2)


def _tc_out_body(s_r, ct_r, h_r, wr_r, b_r, o_r):
    tot = ct_r[:, 0:1] + ct_r[:, 1:2]
    inv = 1.0 / jnp.maximum(tot, 1.0)
    agg = (s_r[0] + s_r[1]) * inv
    hw = jnp.dot(h_r[...], wr_r[...], preferred_element_type=jnp.float32)
    o_r[...] = agg + hw + b_r[...]


def _tc_out(s2, cnt_t, h, wr, b):
    return pl.pallas_call(
        _tc_out_body,
        grid=(N // ROWS_B,),
        in_specs=[pl.BlockSpec((NC, ROWS_B, D), lambda i: (0, i, 0)),
                  pl.BlockSpec((ROWS_B, NC), lambda i: (i, 0)),
                  pl.BlockSpec((ROWS_B, D), lambda i: (i, 0)),
                  pl.BlockSpec((D, D), lambda i: (0, 0)),
                  pl.BlockSpec((1, D), lambda i: (0, 0))],
        out_specs=pl.BlockSpec((ROWS_B, D), lambda i: (i, 0)),
        out_shape=jax.ShapeDtypeStruct((N, D), jnp.float32),
    )(s2, cnt_t, h, wr, b)


def _pad_edges(idx, pad_base):
    tiles = idx.astype(jnp.int32).reshape(NW, EPW)
    padv = pad_base + jnp.arange(PAD, dtype=jnp.int32)
    pad = jnp.broadcast_to(padv, (NW, PAD))
    return jnp.concatenate([tiles, pad], axis=1).reshape(NW, NCH, CH)


def kernel(x, edge_index, W_l1, W_r1, b1, W_l2, W_r2, b2):
    src = _pad_edges(edge_index[0], 0)
    dst = _pad_edges(edge_index[1], N)
    z2 = jnp.zeros((RPT, D), jnp.float32)
    z1 = jnp.zeros((CPT,), jnp.float32)

    cnt = _sc_cnt(dst, z1)
    cnt_t = cnt.T
    y1 = _tc_mm(x, W_l1)
    s1 = _sc_agg(y1, src, dst, z2)
    h, y2 = _tc_mid(s1, cnt_t, x, W_r1, b1.reshape(1, D), W_l2)
    s2 = _sc_agg(y2, src, dst, z2)
    return _tc_out(s2, cnt_t, h, W_r2, b2.reshape(1, D))
